# baseline (speedup 1.0000x reference)
import functools

import numpy as np
import jax
import jax.numpy as jnp
from jax import lax
from jax.experimental import pallas as pl
from jax.experimental.pallas import tpu as pltpu
from jax.experimental.pallas import tpu_sc as plsc

_N = 70
_NX = _N * (_N + 1) // 2
_OUT = _N * _N

_info = plsc.get_sparse_core_info()
_NC, _NS, _L = _info.num_cores, _info.num_subcores, _info.num_lanes
_NC = 1
_NW = _NC * _NS
_CHUNK = 320
_NV = _CHUNK // 16
_LAST = _NW - 1
_TAIL = _OUT - _LAST * _CHUNK


_C_HALF = float(np.sqrt(np.float32(0.5)))


@functools.partial(
    pl.kernel,
    mesh=plsc.VectorSubcoreMesh(core_axis_name="c", subcore_axis_name="s",
                                num_cores=_NC),
    out_type=jax.ShapeDtypeStruct((_OUT,), jnp.float32),
    scratch_types=[
        pltpu.VMEM((_NX,), jnp.float32),
        pltpu.VMEM((_CHUNK,), jnp.float32),
        pltpu.VMEM((_CHUNK,), jnp.int32),
        pltpu.VMEM((_CHUNK,), jnp.float32),
        pltpu.SemaphoreType.DMA,
    ],
    compiler_params=pltpu.CompilerParams(needs_layout_passes=False),
)
def _smat_sc(x_hbm, out_hbm, x_v, o_v, idx_s, scl_s, sem):
    wid = lax.axis_index("s") * _NC + lax.axis_index("c")
    base = wid * _CHUNK
    cp = pltpu.async_copy(x_hbm, x_v, sem)
    lane = lax.iota(jnp.int32, 16)
    pos0 = base + lane
    for v in range(_NV):
        sl = pl.ds(v * 16, 16)
        pos = pos0 + (v * 16)
        i = pos // _N
        j = pos - i * _N
        r = jnp.maximum(i, j)
        c = jnp.minimum(i, j)
        idx_s[sl] = jnp.minimum((r * (r + 1)) // 2 + c, _NX - 1)
        scl_s[sl] = jnp.where(i == j, jnp.float32(1.0), jnp.float32(_C_HALF))
    cp.wait()
    for v in range(_NV):
        sl = pl.ds(v * 16, 16)
        o_v[sl] = plsc.load_gather(x_v, [idx_s[sl]]) * scl_s[sl]

    @pl.when(wid < _LAST)
    def _full():
        pltpu.sync_copy(o_v, out_hbm.at[pl.ds(base, _CHUNK)])

    @pl.when(wid == _LAST)
    def _tail():
        pltpu.sync_copy(o_v.at[pl.ds(0, _TAIL)],
                        out_hbm.at[pl.ds(_LAST * _CHUNK, _TAIL)])


def kernel(x):
    return _smat_sc(x).reshape(_N, _N)


# speedup vs baseline: 1.0252x; 1.0252x over previous
import functools

import numpy as np
import jax
import jax.numpy as jnp
from jax import lax
from jax.experimental import pallas as pl
from jax.experimental.pallas import tpu as pltpu
from jax.experimental.pallas import tpu_sc as plsc

_N = 70
_NX = _N * (_N + 1) // 2
_OUT = _N * _N

_info = plsc.get_sparse_core_info()
_NC, _NS, _L = _info.num_cores, _info.num_subcores, _info.num_lanes
_NC = 1
_NW = _NC * _NS
_CHUNK = 320
_NV = _CHUNK // 16
_LAST = _NW - 1
_TAIL = _OUT - _LAST * _CHUNK


_C_HALF = float(np.sqrt(np.float32(0.5)))
_PAD = _NW * _CHUNK


def _build_table():
    i = np.arange(_N)[:, None]
    j = np.arange(_N)[None, :]
    r = np.maximum(i, j)
    c = np.minimum(i, j)
    idx = np.zeros((_PAD,), np.int32)
    scl = np.zeros((_PAD,), np.float32)
    idx[:_OUT] = (r * (r + 1) // 2 + c).astype(np.int32).reshape(-1)
    scl[:_OUT] = np.where(i == j, 1.0, _C_HALF).astype(np.float32).reshape(-1)
    tbl = np.empty((_NW, 2 * _CHUNK), np.int32)
    tbl[:, :_CHUNK] = idx.reshape(_NW, _CHUNK)
    tbl[:, _CHUNK:] = scl.reshape(_NW, _CHUNK).view(np.int32)
    return tbl.reshape(-1)


_TBL = _build_table()


@functools.partial(
    pl.kernel,
    mesh=plsc.VectorSubcoreMesh(core_axis_name="c", subcore_axis_name="s",
                                num_cores=_NC),
    out_type=jax.ShapeDtypeStruct((_OUT,), jnp.float32),
    scratch_types=[
        pltpu.VMEM((_NX,), jnp.float32),
        pltpu.VMEM((_CHUNK,), jnp.float32),
        pltpu.VMEM((2 * _CHUNK,), jnp.int32),
        pltpu.SemaphoreType.DMA,
        pltpu.SemaphoreType.DMA,
    ],
    compiler_params=pltpu.CompilerParams(needs_layout_passes=False),
)
def _smat_sc(x_hbm, tbl_hbm, out_hbm, x_v, o_v, tbl_v, sem1, sem2):
    wid = lax.axis_index("s") * _NC + lax.axis_index("c")
    base = wid * _CHUNK
    cp1 = pltpu.async_copy(x_hbm, x_v, sem1)
    cp2 = pltpu.async_copy(tbl_hbm.at[pl.ds(wid * 2 * _CHUNK, 2 * _CHUNK)],
                           tbl_v, sem2)
    cp1.wait()
    cp2.wait()
    for v in range(_NV):
        sl = pl.ds(v * 16, 16)
        vals = plsc.load_gather(x_v, [tbl_v[sl]])
        scl = plsc.bitcast(tbl_v[pl.ds(_CHUNK + v * 16, 16)], jnp.float32)
        o_v[sl] = vals * scl

    @pl.when(wid < _LAST)
    def _full():
        pltpu.sync_copy(o_v, out_hbm.at[pl.ds(base, _CHUNK)])

    @pl.when(wid == _LAST)
    def _tail():
        pltpu.sync_copy(o_v.at[pl.ds(0, _TAIL)],
                        out_hbm.at[pl.ds(_LAST * _CHUNK, _TAIL)])


def kernel(x):
    return _smat_sc(x, _TBL).reshape(_N, _N)
